# Initial kernel scaffold; baseline (speedup 1.0000x reference)
import functools

import jax
import jax.numpy as jnp
from jax import lax
from jax.experimental import pallas as pl
from jax.experimental.pallas import tpu as pltpu
from jax.experimental.pallas import tpu_sc as plsc

NC = 2
NS = 16
NW = NC * NS
LANES = 16
CH = 128
G = 64


def _sc_mesh():
    return plsc.VectorSubcoreMesh(core_axis_name="c", subcore_axis_name="s")


def _make_deg_kernel(chunks, n_tab):
    def body(dst_hbm, out_hbm, dst_v, deg_v, sem):
        cid = lax.axis_index("c")
        sid = lax.axis_index("s")
        wid = cid * NS + sid
        cp = pltpu.async_copy(dst_hbm.at[wid], dst_v, sem)
        zeros16 = jnp.zeros((LANES,), jnp.float32)

        def zero_body(i, carry):
            deg_v[pl.ds(i * LANES, LANES)] = zeros16
            return carry

        lax.fori_loop(0, n_tab // LANES, zero_body, 0)
        cp.wait()
        ones16 = jnp.ones((LANES,), jnp.float32)

        def edge_body(j, carry):
            for q in range(CH // LANES):
                idx = dst_v[j, pl.ds(q * LANES, LANES)]
                plsc.addupdate_scatter(deg_v, [idx], ones16)
            return carry

        lax.fori_loop(0, chunks, edge_body, 0)
        pltpu.sync_copy(deg_v, out_hbm.at[wid])

    return pl.kernel(
        body,
        out_type=jax.ShapeDtypeStruct((NW, n_tab), jnp.float32),
        mesh=_sc_mesh(),
        scratch_types=[
            pltpu.VMEM((chunks, CH), jnp.int32),
            pltpu.VMEM((n_tab,), jnp.float32),
            pltpu.SemaphoreType.DMA,
        ],
    )


def _make_edge_kernel(chunks, n_tab, h_dim):
    rows_per_tile = n_tab // NS
    n_full, rem = divmod(rows_per_tile, CH)

    def body(h_hbm, src_hbm, dst_hbm, acc_hbm, src_v, dst_v, rows_v, acc_sh,
             sem, semi):
        cid = lax.axis_index("c")
        sid = lax.axis_index("s")
        wid = cid * NS + sid
        cps = pltpu.async_copy(src_hbm.at[wid], src_v, sem)
        cpd = pltpu.async_copy(dst_hbm.at[wid], dst_v, sem)
        zeros16 = jnp.zeros((LANES,), jnp.float32)

        def zero_body(i, carry):
            for q in range(h_dim // LANES):
                rows_v[i, pl.ds(q * LANES, LANES)] = zeros16
            return carry

        lax.fori_loop(0, CH, zero_body, 0)
        base = sid * rows_per_tile
        for k in range(n_full):
            pltpu.sync_copy(rows_v, acc_sh.at[pl.ds(base + k * CH, CH)])
        if rem:
            pltpu.sync_copy(rows_v.at[pl.ds(0, rem)],
                            acc_sh.at[pl.ds(base + n_full * CH, rem)])
        cps.wait()
        cpd.wait()
        plsc.subcore_barrier()

        def chunk_body(j, carry):
            pltpu.async_copy(h_hbm.at[src_v.at[j]], rows_v, semi).wait()
            pltpu.sync_copy(rows_v, acc_sh.at[dst_v.at[j]], add=True)
            return carry

        lax.fori_loop(0, chunks, chunk_body, 0)
        plsc.subcore_barrier()
        pltpu.sync_copy(acc_sh.at[pl.ds(base, rows_per_tile)],
                        acc_hbm.at[cid, pl.ds(base, rows_per_tile)])

    return pl.kernel(
        body,
        out_type=jax.ShapeDtypeStruct((NC, n_tab, h_dim), jnp.float32),
        mesh=_sc_mesh(),
        scratch_types=[
            pltpu.VMEM((chunks, CH), jnp.int32),
            pltpu.VMEM((chunks, CH), jnp.int32),
            pltpu.VMEM((CH, h_dim), jnp.float32),
            pltpu.VMEM_SHARED((n_tab, h_dim), jnp.float32),
            pltpu.SemaphoreType.DMA,
            pltpu.SemaphoreType.DMA,
        ],
    )


def _scale_body(x_ref, w_ref, degt_ref, h_ref, dinv_ref):
    deg = jnp.sum(degt_ref[...], axis=1, keepdims=True) + 1.0
    dinv = lax.rsqrt(deg)
    dinv_ref[...] = dinv
    h_ref[...] = jnp.dot(x_ref[...], w_ref[...],
                         preferred_element_type=jnp.float32) * dinv


def _mid_body(a0_ref, a1_ref, h_ref, dinv_ref, b_ref, w_ref, out_ref):
    dinv = dinv_ref[...]
    t = (a0_ref[...] + a1_ref[...] + h_ref[...]) * dinv + b_ref[...]
    t = jnp.maximum(t, 0.0)
    out_ref[...] = jnp.dot(t, w_ref[...],
                           preferred_element_type=jnp.float32) * dinv


def _final_body(a0_ref, a1_ref, h_ref, dinv_ref, b_ref, lw_ref, lb_ref,
                batch_ref, out_ref):
    t = (a0_ref[...] + a1_ref[...] + h_ref[...]) * dinv_ref[...] + b_ref[...]
    t = jnp.maximum(t, 0.0)
    s = jnp.dot(t, lw_ref[...], preferred_element_type=jnp.float32)
    gids = lax.broadcasted_iota(jnp.int32, t.shape, 1)
    onehot = (batch_ref[...] == gids).astype(jnp.float32)
    sums = jnp.sum(onehot * s, axis=0, keepdims=True)
    counts = jnp.sum(onehot, axis=0, keepdims=True)
    out_ref[...] = sums / jnp.maximum(counts, 1.0) + lb_ref[...]


def kernel(x, edge_index, batch, W1, b1, W2, b2, lin_W, lin_b):
    n, d = x.shape
    h_dim = W1.shape[1]
    e = edge_index.shape[1]
    pad = LANES
    n_tab = n + pad
    chunks = -(-e // (NW * CH))
    e_pad = chunks * CH * NW

    fill = jnp.full((e_pad - e,), n, jnp.int32)
    src = jnp.concatenate([edge_index[0], fill]).reshape(NW, chunks, CH)
    dst = jnp.concatenate([edge_index[1], fill]).reshape(NW, chunks, CH)
    x_pad = jnp.concatenate([x, jnp.zeros((pad, d), x.dtype)])
    batch_col = jnp.concatenate(
        [batch, jnp.full((pad,), G, jnp.int32)]).reshape(n_tab, 1)
    b1_row = b1.reshape(1, h_dim)
    b2_row = b2.reshape(1, h_dim)
    lb_2d = lin_b.reshape(1, 1)

    deg_parts = _make_deg_kernel(chunks, n_tab)(dst)
    degt = deg_parts.T

    h1p, dinv = pl.pallas_call(
        _scale_body,
        out_shape=(
            jax.ShapeDtypeStruct((n_tab, h_dim), jnp.float32),
            jax.ShapeDtypeStruct((n_tab, 1), jnp.float32),
        ),
    )(x_pad, W1, degt)

    edge_kernel = _make_edge_kernel(chunks, n_tab, h_dim)

    acc1 = edge_kernel(h1p, src, dst)
    h2p = pl.pallas_call(
        _mid_body,
        out_shape=jax.ShapeDtypeStruct((n_tab, h_dim), jnp.float32),
    )(acc1[0], acc1[1], h1p, dinv, b1_row, W2)

    acc2 = edge_kernel(h2p, src, dst)
    logits = pl.pallas_call(
        _final_body,
        out_shape=jax.ShapeDtypeStruct((1, G), jnp.float32),
    )(acc2[0], acc2[1], h2p, dinv, b2_row, lin_W, lb_2d, batch_col)

    return logits.reshape(-1)


# speedup vs baseline: 18.7701x; 18.7701x over previous
import functools

import jax
import jax.numpy as jnp
from jax import lax
from jax.experimental import pallas as pl
from jax.experimental.pallas import tpu as pltpu
from jax.experimental.pallas import tpu_sc as plsc

NC = 2
NS = 16
NW = NC * NS
LANES = 16
CH = 128
G = 64


def _sc_mesh():
    return plsc.VectorSubcoreMesh(core_axis_name="c", subcore_axis_name="s")


def _make_deg_kernel(chunks, n_tab):
    def body(dst_hbm, out_hbm, dst_v, deg_v, sem):
        cid = lax.axis_index("c")
        sid = lax.axis_index("s")
        wid = cid * NS + sid
        cp = pltpu.async_copy(dst_hbm.at[wid], dst_v, sem)
        zeros16 = jnp.zeros((LANES,), jnp.float32)

        def zero_body(i, carry):
            deg_v[pl.ds(i * LANES, LANES)] = zeros16
            return carry

        lax.fori_loop(0, n_tab // LANES, zero_body, 0)
        cp.wait()
        ones16 = jnp.ones((LANES,), jnp.float32)

        def edge_body(j, carry):
            for q in range(CH // LANES):
                idx = dst_v[j, pl.ds(q * LANES, LANES)]
                plsc.addupdate_scatter(deg_v, [idx], ones16)
            return carry

        lax.fori_loop(0, chunks, edge_body, 0)
        pltpu.sync_copy(deg_v, out_hbm.at[wid])

    return pl.kernel(
        body,
        out_type=jax.ShapeDtypeStruct((NW, n_tab), jnp.float32),
        mesh=_sc_mesh(),
        compiler_params=pltpu.CompilerParams(needs_layout_passes=False, use_tc_tiling_on_sc=False),
        scratch_types=[
            pltpu.VMEM((chunks, CH), jnp.int32),
            pltpu.VMEM((n_tab,), jnp.float32),
            pltpu.SemaphoreType.DMA,
        ],
    )


def _make_edge_kernel(chunks, n_tab, h_dim):
    rows_per_tile = n_tab // NS
    n_full, rem = divmod(rows_per_tile, CH)

    def body(h_hbm, src_hbm, dst_hbm, acc_hbm, src_v, dst_v, rows_v, acc_sh,
             sem, semi):
        cid = lax.axis_index("c")
        sid = lax.axis_index("s")
        wid = cid * NS + sid
        cps = pltpu.async_copy(src_hbm.at[wid], src_v, sem)
        cpd = pltpu.async_copy(dst_hbm.at[wid], dst_v, sem)
        zeros16 = jnp.zeros((LANES,), jnp.float32)

        def zero_body(i, carry):
            for q in range(h_dim // LANES):
                rows_v[i, pl.ds(q * LANES, LANES)] = zeros16
            return carry

        lax.fori_loop(0, CH, zero_body, 0)
        base = sid * rows_per_tile
        for k in range(n_full):
            pltpu.sync_copy(rows_v, acc_sh.at[pl.ds(base + k * CH, CH)])
        if rem:
            pltpu.sync_copy(rows_v.at[pl.ds(0, rem)],
                            acc_sh.at[pl.ds(base + n_full * CH, rem)])
        cps.wait()
        cpd.wait()
        plsc.subcore_barrier()

        def chunk_body(j, carry):
            pltpu.async_copy(h_hbm.at[src_v.at[j]], rows_v, semi).wait()
            pltpu.sync_copy(rows_v, acc_sh.at[dst_v.at[j]], add=True)
            return carry

        lax.fori_loop(0, chunks, chunk_body, 0)
        plsc.subcore_barrier()
        pltpu.sync_copy(acc_sh.at[pl.ds(base, rows_per_tile)],
                        acc_hbm.at[cid, pl.ds(base, rows_per_tile)])

    return pl.kernel(
        body,
        out_type=jax.ShapeDtypeStruct((NC, n_tab, h_dim), jnp.float32),
        mesh=_sc_mesh(),
        compiler_params=pltpu.CompilerParams(needs_layout_passes=False, use_tc_tiling_on_sc=False),
        scratch_types=[
            pltpu.VMEM((chunks, CH), jnp.int32),
            pltpu.VMEM((chunks, CH), jnp.int32),
            pltpu.VMEM((CH, h_dim), jnp.float32),
            pltpu.VMEM_SHARED((n_tab, h_dim), jnp.float32),
            pltpu.SemaphoreType.DMA,
            pltpu.SemaphoreType.DMA,
        ],
    )


def _scale_body(x_ref, w_ref, degt_ref, h_ref, dinv_ref):
    deg = jnp.sum(degt_ref[...], axis=1, keepdims=True) + 1.0
    dinv = lax.rsqrt(deg)
    dinv_ref[...] = dinv
    h_ref[...] = jnp.dot(x_ref[...], w_ref[...],
                         preferred_element_type=jnp.float32) * dinv


def _mid_body(a0_ref, a1_ref, h_ref, dinv_ref, b_ref, w_ref, out_ref):
    dinv = dinv_ref[...]
    t = (a0_ref[...] + a1_ref[...] + h_ref[...]) * dinv + b_ref[...]
    t = jnp.maximum(t, 0.0)
    out_ref[...] = jnp.dot(t, w_ref[...],
                           preferred_element_type=jnp.float32) * dinv


def _final_body(a0_ref, a1_ref, h_ref, dinv_ref, b_ref, lw_ref, lb_ref,
                batch_ref, out_ref):
    t = (a0_ref[...] + a1_ref[...] + h_ref[...]) * dinv_ref[...] + b_ref[...]
    t = jnp.maximum(t, 0.0)
    s = jnp.dot(t, lw_ref[...], preferred_element_type=jnp.float32)
    gids = lax.broadcasted_iota(jnp.int32, t.shape, 1)
    onehot = (batch_ref[...] == gids).astype(jnp.float32)
    sums = jnp.sum(onehot * s, axis=0, keepdims=True)
    counts = jnp.sum(onehot, axis=0, keepdims=True)
    out_ref[...] = sums / jnp.maximum(counts, 1.0) + lb_ref[...]


def kernel(x, edge_index, batch, W1, b1, W2, b2, lin_W, lin_b):
    n, d = x.shape
    h_dim = W1.shape[1]
    e = edge_index.shape[1]
    n_tab = -(-(n + 1) // (NS * 8)) * (NS * 8)
    pad = n_tab - n
    chunks = -(-e // (NW * CH))
    e_pad = chunks * CH * NW

    fill = jnp.full((e_pad - e,), n, jnp.int32)
    src = jnp.concatenate([edge_index[0], fill]).reshape(NW, chunks, CH)
    dst = jnp.concatenate([edge_index[1], fill]).reshape(NW, chunks, CH)
    x_pad = jnp.concatenate([x, jnp.zeros((pad, d), x.dtype)])
    batch_col = jnp.concatenate(
        [batch, jnp.full((pad,), G, jnp.int32)]).reshape(n_tab, 1)
    b1_row = b1.reshape(1, h_dim)
    b2_row = b2.reshape(1, h_dim)
    lb_2d = lin_b.reshape(1, 1)

    deg_parts = _make_deg_kernel(chunks, n_tab)(dst)
    degt = deg_parts.T

    h1p, dinv = pl.pallas_call(
        _scale_body,
        out_shape=(
            jax.ShapeDtypeStruct((n_tab, h_dim), jnp.float32),
            jax.ShapeDtypeStruct((n_tab, 1), jnp.float32),
        ),
    )(x_pad, W1, degt)

    edge_kernel = _make_edge_kernel(chunks, n_tab, h_dim)

    acc1 = edge_kernel(h1p, src, dst)
    h2p = pl.pallas_call(
        _mid_body,
        out_shape=jax.ShapeDtypeStruct((n_tab, h_dim), jnp.float32),
    )(acc1[0], acc1[1], h1p, dinv, b1_row, W2)

    acc2 = edge_kernel(h2p, src, dst)
    logits = pl.pallas_call(
        _final_body,
        out_shape=jax.ShapeDtypeStruct((1, G), jnp.float32),
    )(acc2[0], acc2[1], h2p, dinv, b2_row, lin_W, lb_2d, batch_col)

    return logits.reshape(-1)


# speedup vs baseline: 19.7341x; 1.0514x over previous
import functools

import jax
import jax.numpy as jnp
from jax import lax
from jax.experimental import pallas as pl
from jax.experimental.pallas import tpu as pltpu
from jax.experimental.pallas import tpu_sc as plsc

NC = 2
NS = 16
NW = NC * NS
LANES = 16
CH = 128
G = 64


def _sc_mesh():
    return plsc.VectorSubcoreMesh(core_axis_name="c", subcore_axis_name="s")


def _make_deg_kernel(chunks, n_tab):
    def body(dst_hbm, out_hbm, dst_v, deg_v, sem):
        cid = lax.axis_index("c")
        sid = lax.axis_index("s")
        wid = cid * NS + sid
        cp = pltpu.async_copy(dst_hbm.at[wid], dst_v, sem)
        zeros16 = jnp.zeros((LANES,), jnp.float32)

        def zero_body(i, carry):
            deg_v[pl.ds(i * LANES, LANES)] = zeros16
            return carry

        lax.fori_loop(0, n_tab // LANES, zero_body, 0)
        cp.wait()
        ones16 = jnp.ones((LANES,), jnp.float32)

        def edge_body(j, carry):
            for q in range(CH // LANES):
                idx = dst_v[j, pl.ds(q * LANES, LANES)]
                plsc.addupdate_scatter(deg_v, [idx], ones16)
            return carry

        lax.fori_loop(0, chunks, edge_body, 0)
        pltpu.sync_copy(deg_v, out_hbm.at[wid])

    return pl.kernel(
        body,
        out_type=jax.ShapeDtypeStruct((NW, n_tab), jnp.float32),
        mesh=_sc_mesh(),
        compiler_params=pltpu.CompilerParams(needs_layout_passes=False, use_tc_tiling_on_sc=False),
        scratch_types=[
            pltpu.VMEM((chunks, CH), jnp.int32),
            pltpu.VMEM((n_tab,), jnp.float32),
            pltpu.SemaphoreType.DMA,
        ],
    )


NBUF = 4


def _make_edge_kernel(chunks, n_tab, h_dim):
    rows_per_tile = n_tab // NS
    n_full, rem = divmod(rows_per_tile, CH)
    n_batches = chunks // NBUF
    assert n_batches % 2 == 0

    def body(h_hbm, src_hbm, dst_hbm, acc_hbm, src_v, dst_v,
             a0, a1, a2, a3, b0, b1, b2, b3, acc_sh, sem, semga, semgb):
        bufs_a = (a0, a1, a2, a3)
        bufs_b = (b0, b1, b2, b3)
        cid = lax.axis_index("c")
        sid = lax.axis_index("s")
        wid = cid * NS + sid
        cps = pltpu.async_copy(src_hbm.at[wid], src_v, sem)
        cpd = pltpu.async_copy(dst_hbm.at[wid], dst_v, sem)
        zeros16 = jnp.zeros((LANES,), jnp.float32)

        def zero_body(i, carry):
            for q in range(h_dim // LANES):
                a0[i, pl.ds(q * LANES, LANES)] = zeros16
            return carry

        lax.fori_loop(0, CH, zero_body, 0)
        base = sid * rows_per_tile
        for k in range(n_full):
            pltpu.sync_copy(a0, acc_sh.at[pl.ds(base + k * CH, CH)])
        if rem:
            pltpu.sync_copy(a0.at[pl.ds(0, rem)],
                            acc_sh.at[pl.ds(base + n_full * CH, rem)])
        cps.wait()
        cpd.wait()
        plsc.subcore_barrier()

        for b in range(NBUF):
            pltpu.async_copy(h_hbm.at[src_v.at[b]], bufs_a[b], semga)
        for b in range(NBUF):
            pltpu.async_copy(h_hbm.at[src_v.at[NBUF + b]], bufs_b[b], semgb)

        def pair_body(i, carry):
            for bufs, semg, half in ((bufs_a, semga, 0), (bufs_b, semgb, 1)):
                bi = i * 2 + half
                j0 = bi * NBUF
                for b in range(NBUF):
                    pltpu.make_async_copy(h_hbm.at[src_v.at[j0 + b]],
                                          bufs[b], semg).wait()
                for b in range(NBUF):
                    pltpu.sync_copy(bufs[b], acc_sh.at[dst_v.at[j0 + b]],
                                    add=True)

                @pl.when(bi + 2 < n_batches)
                def _():
                    for b in range(NBUF):
                        pltpu.async_copy(
                            h_hbm.at[src_v.at[j0 + 2 * NBUF + b]],
                            bufs[b], semg)
            return carry

        lax.fori_loop(0, n_batches // 2, pair_body, 0)
        plsc.subcore_barrier()
        pltpu.sync_copy(acc_sh.at[pl.ds(base, rows_per_tile)],
                        acc_hbm.at[cid, pl.ds(base, rows_per_tile)])

    return pl.kernel(
        body,
        out_type=jax.ShapeDtypeStruct((NC, n_tab, h_dim), jnp.float32),
        mesh=_sc_mesh(),
        compiler_params=pltpu.CompilerParams(needs_layout_passes=False, use_tc_tiling_on_sc=False),
        scratch_types=[
            pltpu.VMEM((chunks, CH), jnp.int32),
            pltpu.VMEM((chunks, CH), jnp.int32),
        ] + [pltpu.VMEM((CH, h_dim), jnp.float32) for _ in range(2 * NBUF)]
        + [
            pltpu.VMEM_SHARED((n_tab, h_dim), jnp.float32),
            pltpu.SemaphoreType.DMA,
            pltpu.SemaphoreType.DMA,
            pltpu.SemaphoreType.DMA,
        ],
    )


def _scale_body(x_ref, w_ref, degt_ref, h_ref, dinv_ref):
    deg = jnp.sum(degt_ref[...], axis=1, keepdims=True) + 1.0
    dinv = lax.rsqrt(deg)
    dinv_ref[...] = dinv
    h_ref[...] = jnp.dot(x_ref[...], w_ref[...],
                         preferred_element_type=jnp.float32,
                         precision=lax.Precision.HIGHEST) * dinv


def _mid_body(a0_ref, a1_ref, h_ref, dinv_ref, b_ref, w_ref, out_ref):
    dinv = dinv_ref[...]
    t = (a0_ref[...] + a1_ref[...] + h_ref[...]) * dinv + b_ref[...]
    t = jnp.maximum(t, 0.0)
    out_ref[...] = jnp.dot(t, w_ref[...],
                           preferred_element_type=jnp.float32,
                         precision=lax.Precision.HIGHEST) * dinv


def _final_body(a0_ref, a1_ref, h_ref, dinv_ref, b_ref, lw_ref, lb_ref,
                batch_ref, out_ref):
    t = (a0_ref[...] + a1_ref[...] + h_ref[...]) * dinv_ref[...] + b_ref[...]
    t = jnp.maximum(t, 0.0)
    s = jnp.dot(t, lw_ref[...], preferred_element_type=jnp.float32,
                         precision=lax.Precision.HIGHEST)
    gids = lax.broadcasted_iota(jnp.int32, t.shape, 1)
    onehot = (batch_ref[...] == gids).astype(jnp.float32)
    sums = jnp.sum(onehot * s, axis=0, keepdims=True)
    counts = jnp.sum(onehot, axis=0, keepdims=True)
    out_ref[...] = sums / jnp.maximum(counts, 1.0) + lb_ref[...]


def kernel(x, edge_index, batch, W1, b1, W2, b2, lin_W, lin_b):
    n, d = x.shape
    h_dim = W1.shape[1]
    e = edge_index.shape[1]
    n_tab = -(-(n + 1) // (NS * 8)) * (NS * 8)
    pad = n_tab - n
    chunks = -(-e // (NW * CH))
    chunks = -(-chunks // (2 * NBUF)) * (2 * NBUF)
    e_pad = chunks * CH * NW

    fill = jnp.full((e_pad - e,), n, jnp.int32)
    src = jnp.concatenate([edge_index[0], fill]).reshape(NW, chunks, CH)
    dst = jnp.concatenate([edge_index[1], fill]).reshape(NW, chunks, CH)
    x_pad = jnp.concatenate([x, jnp.zeros((pad, d), x.dtype)])
    batch_col = jnp.concatenate(
        [batch, jnp.full((pad,), G, jnp.int32)]).reshape(n_tab, 1)
    b1_row = b1.reshape(1, h_dim)
    b2_row = b2.reshape(1, h_dim)
    lb_2d = lin_b.reshape(1, 1)

    deg_parts = _make_deg_kernel(chunks, n_tab)(dst)
    degt = deg_parts.T

    h1p, dinv = pl.pallas_call(
        _scale_body,
        out_shape=(
            jax.ShapeDtypeStruct((n_tab, h_dim), jnp.float32),
            jax.ShapeDtypeStruct((n_tab, 1), jnp.float32),
        ),
    )(x_pad, W1, degt)

    edge_kernel = _make_edge_kernel(chunks, n_tab, h_dim)

    acc1 = edge_kernel(h1p, src, dst)
    h2p = pl.pallas_call(
        _mid_body,
        out_shape=jax.ShapeDtypeStruct((n_tab, h_dim), jnp.float32),
    )(acc1[0], acc1[1], h1p, dinv, b1_row, W2)

    acc2 = edge_kernel(h2p, src, dst)
    logits = pl.pallas_call(
        _final_body,
        out_shape=jax.ShapeDtypeStruct((1, G), jnp.float32),
    )(acc2[0], acc2[1], h2p, dinv, b2_row, lin_W, lb_2d, batch_col)

    return logits.reshape(-1)
